# baseline (speedup 1.0000x reference)
import jax
import jax.numpy as jnp
from jax.experimental import pallas as pl
from jax.experimental.pallas import tpu as pltpu

_ALPHA = 0.2
_TBLK = 256
_NBUF = 2


def _tile_copy(adj_hbm, buf, sem, k):
    return pltpu.make_async_copy(
        adj_hbm.at[pl.ds(k * _TBLK, _TBLK), :], buf.at[k % _NBUF],
        sem.at[k % _NBUF])


def _out_copy(out_hbm, obuf, osem, k):
    return pltpu.make_async_copy(
        obuf.at[k % _NBUF], out_hbm.at[:, pl.ds(k * _TBLK, _TBLK), :],
        osem.at[k % _NBUF])


def _gat_kernel(x_ref, adj_hbm, w_ref, a_ref, out_hbm, buf, obuf, sem, osem):
    Bb, N, in_f = x_ref.shape
    out_f = w_ref.shape[1]
    nt = N // _TBLK

    for k in range(_NBUF):
        _tile_copy(adj_hbm, buf, sem, k).start()

    x2 = x_ref[...].reshape(Bb * N, in_f)
    h = jnp.dot(x2, w_ref[...], preferred_element_type=jnp.float32)
    h = jnp.where(jnp.isnan(h), 0.0, h)
    a = a_ref[...]
    na1 = -a[:, :out_f]
    na2 = -a[:, out_f:]
    fneg = jax.lax.dot_general(h, na1, (((1,), (1,)), ((), ())),
                               preferred_element_type=jnp.float32)
    gneg = jax.lax.dot_general(na2, h, (((1,), (1,)), ((), ())),
                               preferred_element_type=jnp.float32)
    ones = jnp.ones((Bb * N, 8), jnp.float32)
    haug = jnp.concatenate([h, ones], axis=1).astype(jnp.bfloat16)

    for k in range(nt):
        _tile_copy(adj_hbm, buf, sem, k).wait()
        adj_blk = buf[k % _NBUF]
        es = []
        for b in range(Bb):
            fn = fneg[b * N + k * _TBLK:b * N + (k + 1) * _TBLK, :]
            gn = gneg[:, b * N:(b + 1) * N]
            t = fn + gn
            e = adj_blk * jnp.exp(jnp.minimum(t, _ALPHA * t))
            es.append(e.astype(jnp.bfloat16))
        if k + _NBUF < nt:
            _tile_copy(adj_hbm, buf, sem, k + _NBUF).start()
        if k >= _NBUF:
            _out_copy(out_hbm, obuf, osem, k - _NBUF).wait()
        for b in range(Bb):
            hb = haug[b * N:(b + 1) * N, :]
            p = jnp.dot(es[b], hb,
                        preferred_element_type=jnp.float32)
            rowsum = p[:, out_f:out_f + 1]
            recip = jnp.where(rowsum != 0, 1.0 / rowsum, 1.0)
            hp = p[:, :out_f] * recip
            hp = jnp.where(jnp.isnan(hp), 0.0, hp)
            hp = jnp.where(hp > 0, hp, jnp.exp(jnp.minimum(hp, 0.0)) - 1.0)
            obuf[k % _NBUF, b, :, :] = hp
        _out_copy(out_hbm, obuf, osem, k).start()

    for k in range(max(nt - _NBUF, 0), nt):
        _out_copy(out_hbm, obuf, osem, k).wait()


def kernel(inputBatch, adj, W, a):
    Bb, N, in_f = inputBatch.shape
    out_f = W.shape[1]
    return pl.pallas_call(
        _gat_kernel,
        in_specs=[
            pl.BlockSpec((Bb, N, in_f), lambda: (0, 0, 0)),
            pl.BlockSpec(memory_space=pltpu.MemorySpace.HBM),
            pl.BlockSpec((in_f, out_f), lambda: (0, 0)),
            pl.BlockSpec((1, 2 * out_f), lambda: (0, 0)),
        ],
        out_specs=pl.BlockSpec(memory_space=pltpu.MemorySpace.HBM),
        out_shape=jax.ShapeDtypeStruct((Bb, N, out_f), jnp.float32),
        scratch_shapes=[
            pltpu.VMEM((_NBUF, _TBLK, N), jnp.float32),
            pltpu.VMEM((_NBUF, Bb, _TBLK, out_f), jnp.float32),
            pltpu.SemaphoreType.DMA((_NBUF,)),
            pltpu.SemaphoreType.DMA((_NBUF,)),
        ],
    )(inputBatch, adj, W, a)
